# baseline (speedup 1.0000x reference)
import functools

import jax
import jax.numpy as jnp
from jax import lax
from jax.experimental import pallas as pl
from jax.experimental.pallas import tpu as pltpu
from jax.experimental.pallas import tpu_sc as plsc

B, T, F = 16, 4096, 256
CH_T = 32
CPR = T // CH_T
NC, NS, L = 2, 16, 16
NW = NC * NS
KPW = CPR // NW
NBUF = 4

_mesh = plsc.VectorSubcoreMesh(core_axis_name="c", subcore_axis_name="s")


@functools.partial(
    pl.kernel,
    out_type=jax.ShapeDtypeStruct((NW, L), jnp.float32),
    mesh=_mesh,
    compiler_params=pltpu.CompilerParams(
        needs_layout_passes=False,
        use_tc_tiling_on_sc=True,
    ),
    scratch_types=[
        pltpu.VMEM((L,), jnp.int32),
        pltpu.SMEM((L,), jnp.int32),
        pltpu.VMEM((NBUF, CH_T, F), jnp.float32),
        pltpu.VMEM((L,), jnp.float32),
    ] + [pltpu.SemaphoreType.DMA] * NBUF,
)
def _ragged_diff(x_hbm, len_hbm, out_hbm, len_v, len_s, bufs, accv, *sems):
    w = lax.axis_index("s") * NC + lax.axis_index("c")
    pltpu.sync_copy(len_hbm, len_v)
    nv = len_v[...]
    lane = lax.broadcasted_iota(jnp.int32, (L,), 0)

    for i in range(B):
        len_s[i] = jnp.max(jnp.where(lane == i, nv, 0))

    accv[...] = jnp.zeros((L,), jnp.float32)

    NSLOT = B * KPW

    def slot(s):
        i = s // KPW
        k = s % KPW
        rot = (w + i * 7) & (NW - 1)
        t0 = (k * NW + rot) * CH_T
        live = t0 < len_s[i]
        return i, t0, live

    def start(s, q):
        i, t0, live = slot(s)

        @pl.when(live)
        def _():
            pltpu.async_copy(x_hbm.at[i, pl.ds(t0, CH_T), :],
                             bufs.at[q], sems[q])

    def finish(s, q):
        i, t0, live = slot(s)

        @pl.when(live)
        def _():
            pltpu.make_async_copy(x_hbm.at[i, pl.ds(t0, CH_T), :],
                                  bufs.at[q], sems[q]).wait()

            def vbody(t, accs):
                out = []
                for j in range(F // L):
                    a = bufs[q, t, pl.ds(j * L, L)]
                    out.append(accs[j] + (a - a))
                return tuple(out)

            zeros = jnp.zeros((L,), jnp.float32)
            accs = lax.fori_loop(0, CH_T, vbody, (zeros,) * (F // L),
                                 unroll=4)
            s_ = accs[0]
            for j in range(1, F // L):
                s_ = s_ + accs[j]
            accv[...] = accv[...] + s_

    for s in range(NBUF - 1):
        start(s, s)

    def quad_body(m, carry):
        s = m * NBUF
        for q in range(NBUF):
            sp = s + q + NBUF - 1

            @pl.when(sp < NSLOT)
            def _(sp=sp, q=q):
                start(sp, (q + NBUF - 1) % NBUF)

            finish(s + q, q)
        return carry

    lax.fori_loop(0, NSLOT // NBUF, quad_body, 0)
    pltpu.sync_copy(accv, out_hbm.at[w])


def kernel(batch_input, seq_length):
    partials = _ragged_diff(batch_input, seq_length)
    total = jnp.sum(partials)
    return (total == 0.0).astype(jnp.float32).reshape(1)


# speedup vs baseline: 1.0167x; 1.0167x over previous
import functools

import jax
import jax.numpy as jnp
from jax import lax
from jax.experimental import pallas as pl
from jax.experimental.pallas import tpu as pltpu
from jax.experimental.pallas import tpu_sc as plsc

B, T, F = 16, 4096, 256
CH_T = 64
CPR = T // CH_T
NC, NS, L = 2, 16, 16
NW = NC * NS
KPW = CPR // NW
NBUF = 4

_mesh = plsc.VectorSubcoreMesh(core_axis_name="c", subcore_axis_name="s")


@functools.partial(
    pl.kernel,
    out_type=jax.ShapeDtypeStruct((NW, L), jnp.float32),
    mesh=_mesh,
    compiler_params=pltpu.CompilerParams(
        needs_layout_passes=False,
        use_tc_tiling_on_sc=True,
    ),
    scratch_types=[
        pltpu.VMEM((L,), jnp.int32),
        pltpu.SMEM((L,), jnp.int32),
        pltpu.VMEM((NBUF, CH_T, F), jnp.float32),
        pltpu.VMEM((L,), jnp.float32),
    ] + [pltpu.SemaphoreType.DMA] * NBUF,
)
def _ragged_diff(x_hbm, len_hbm, out_hbm, len_v, len_s, bufs, accv, *sems):
    w = lax.axis_index("s") * NC + lax.axis_index("c")
    pltpu.sync_copy(len_hbm, len_v)
    nv = len_v[...]
    lane = lax.broadcasted_iota(jnp.int32, (L,), 0)

    for i in range(B):
        len_s[i] = jnp.max(jnp.where(lane == i, nv, 0))

    accv[...] = jnp.zeros((L,), jnp.float32)

    NSLOT = B * KPW

    def slot(s):
        i = s // KPW
        k = s % KPW
        rot = (w + i * 7) & (NW - 1)
        t0 = (k * NW + rot) * CH_T
        live = t0 < len_s[i]
        return i, t0, live

    def start(s, q):
        i, t0, live = slot(s)

        @pl.when(live)
        def _():
            pltpu.async_copy(x_hbm.at[i, pl.ds(t0, CH_T), :],
                             bufs.at[q], sems[q])

    def finish(s, q):
        i, t0, live = slot(s)

        @pl.when(live)
        def _():
            pltpu.make_async_copy(x_hbm.at[i, pl.ds(t0, CH_T), :],
                                  bufs.at[q], sems[q]).wait()

            def vbody(t, accs):
                out = []
                for j in range(F // L):
                    a = bufs[q, t, pl.ds(j * L, L)]
                    out.append(accs[j] + (a - a))
                return tuple(out)

            zeros = jnp.zeros((L,), jnp.float32)
            accs = lax.fori_loop(0, CH_T, vbody, (zeros,) * (F // L),
                                 unroll=4)
            s_ = accs[0]
            for j in range(1, F // L):
                s_ = s_ + accs[j]
            accv[...] = accv[...] + s_

    for s in range(NBUF - 1):
        start(s, s)

    def quad_body(m, carry):
        s = m * NBUF
        for q in range(NBUF):
            sp = s + q + NBUF - 1

            @pl.when(sp < NSLOT)
            def _(sp=sp, q=q):
                start(sp, (q + NBUF - 1) % NBUF)

            finish(s + q, q)
        return carry

    lax.fori_loop(0, NSLOT // NBUF, quad_body, 0)
    pltpu.sync_copy(accv, out_hbm.at[w])


def kernel(batch_input, seq_length):
    partials = _ragged_diff(batch_input, seq_length)
    total = jnp.sum(partials)
    return (total == 0.0).astype(jnp.float32).reshape(1)


# speedup vs baseline: 1.0303x; 1.0134x over previous
import functools

import jax
import jax.numpy as jnp
from jax import lax
from jax.experimental import pallas as pl
from jax.experimental.pallas import tpu as pltpu
from jax.experimental.pallas import tpu_sc as plsc

B, T, F = 16, 4096, 256
CH_T = 64
CPR = T // CH_T
NC, NS, L = 2, 16, 16
NW = NC * NS
KPW = CPR // NW
NBUF = 4

_mesh = plsc.VectorSubcoreMesh(core_axis_name="c", subcore_axis_name="s")


@functools.partial(
    pl.kernel,
    out_type=jax.ShapeDtypeStruct((NW, L), jnp.float32),
    mesh=_mesh,
    compiler_params=pltpu.CompilerParams(
        needs_layout_passes=False,
        use_tc_tiling_on_sc=True,
    ),
    scratch_types=[
        pltpu.VMEM((L,), jnp.int32),
        pltpu.SMEM((L,), jnp.int32),
        pltpu.VMEM((NBUF, CH_T, F), jnp.float32),
        pltpu.VMEM((L,), jnp.float32),
    ] + [pltpu.SemaphoreType.DMA] * NBUF,
)
def _ragged_diff(x_hbm, len_hbm, out_hbm, len_v, len_s, bufs, accv, *sems):
    w = lax.axis_index("s") * NC + lax.axis_index("c")
    pltpu.sync_copy(len_hbm, len_v)
    nv = len_v[...]
    lane = lax.broadcasted_iota(jnp.int32, (L,), 0)

    for i in range(B):
        len_s[i] = jnp.max(jnp.where(lane == i, nv, 0))

    accv[...] = jnp.zeros((L,), jnp.float32)

    NSLOT = B * KPW

    def slot(s):
        i = s // KPW
        k = s % KPW
        rot = (w + i * 7) & (NW - 1)
        t0 = (k * NW + rot) * CH_T
        live = t0 < len_s[i]
        return i, t0, live

    def start(s, q):
        i, t0, live = slot(s)

        @pl.when(live)
        def _():
            pltpu.async_copy(x_hbm.at[i, pl.ds(t0, CH_T), :],
                             bufs.at[q], sems[q])

    def finish(s, q):
        i, t0, live = slot(s)

        @pl.when(live)
        def _():
            pltpu.make_async_copy(x_hbm.at[i, pl.ds(t0, CH_T), :],
                                  bufs.at[q], sems[q]).wait()

            def vbody(t, accs):
                out = []
                for j in range(F // L):
                    a = bufs[q, t, pl.ds(j * L, L)]
                    out.append(accs[j] + (a - a))
                return tuple(out)

            zeros = jnp.zeros((L,), jnp.float32)
            accs = lax.fori_loop(0, CH_T, vbody, (zeros,) * (F // L),
                                 unroll=2)
            s_ = accs[0]
            for j in range(1, F // L):
                s_ = s_ + accs[j]
            accv[...] = accv[...] + s_

    for s in range(NBUF - 1):
        start(s, s)

    def quad_body(m, carry):
        s = m * NBUF
        for q in range(NBUF):
            sp = s + q + NBUF - 1

            @pl.when(sp < NSLOT)
            def _(sp=sp, q=q):
                start(sp, (q + NBUF - 1) % NBUF)

            finish(s + q, q)
        return carry

    lax.fori_loop(0, NSLOT // NBUF, quad_body, 0)
    pltpu.sync_copy(accv, out_hbm.at[w])


def kernel(batch_input, seq_length):
    partials = _ragged_diff(batch_input, seq_length)
    total = jnp.sum(partials)
    return (total == 0.0).astype(jnp.float32).reshape(1)
